# Initial kernel scaffold; baseline (speedup 1.0000x reference)
import jax
import jax.numpy as jnp
from jax.experimental import pallas as pl


def kernel(roi_locs, roi_scores, rois):
    raise NotImplementedError("write your pallas kernel here")


# speedup vs baseline: 92.4432x; 92.4432x over previous
import functools

import jax
import jax.numpy as jnp
from jax import lax
from jax.experimental import pallas as pl
from jax.experimental.pallas import tpu as pltpu
from jax.experimental.pallas import tpu_sc as plsc

_N = 5000
_NPAD = 5120
_NVEC = _NPAD // 16
_NCLS = 20
_NW = 32
_SLAB = 25600
_NMS_T = 0.3
_SCORE_T = 0.05
_NC = 2
_IMG_H, _IMG_W = 600.0, 800.0


def _sc_nms_body(probt_hbm, py0_h, px0_h, py1_h, px1_h, par_h, out_hbm,
                 prob_v, y0_v, x0_v, y1_v, x1_v, ar_v,
                 ci, cs, cy0, cx0, cy1, cx1, car, slab, stf, sti):
    w = lax.axis_index("s") * _NC + lax.axis_index("c")
    iota = lax.iota(jnp.int32, 16)
    lane0 = iota == 0
    zf = jnp.zeros((16,), jnp.float32)
    neg = jnp.full((16,), -1.0, jnp.float32)
    sixteen = jnp.full((16,), 16, jnp.int32)
    big = jnp.full((16,), 2 ** 30, jnp.int32)

    pltpu.sync_copy(probt_hbm.at[w], prob_v)
    pltpu.sync_copy(py0_h, y0_v)
    pltpu.sync_copy(px0_h, x0_v)
    pltpu.sync_copy(py1_h, y1_v)
    pltpu.sync_copy(px1_h, x1_v)
    pltpu.sync_copy(par_h, ar_v)

    def zero_body(v, _):
        slab[pl.ds(v * 16, 16)] = zf
        return 0

    lax.fori_loop(0, _SLAB // 16, zero_body, 0)

    def sent_body(v, _):
        cs[pl.ds(v * 16, 16)] = neg
        return 0

    lax.fori_loop(0, _NVEC, sent_body, 0)

    sti[pl.ds(0, 16)] = jnp.zeros((16,), jnp.int32)
    sti[pl.ds(16, 16)] = iota

    def comp_body(v, _):
        p = prob_v[pl.ds(v * 16, 16)]
        m = p > _SCORE_T
        pos = plsc.cumsum(m.astype(jnp.int32))
        basev = sti[pl.ds(0, 16)]
        dst = basev + pos - 1
        gi = sti[pl.ds(16, 16)]
        plsc.store_scatter(ci, [dst], gi, mask=m)
        plsc.store_scatter(cs, [dst], p, mask=m)
        plsc.store_scatter(cy0, [dst], y0_v[pl.ds(v * 16, 16)], mask=m)
        plsc.store_scatter(cx0, [dst], x0_v[pl.ds(v * 16, 16)], mask=m)
        plsc.store_scatter(cy1, [dst], y1_v[pl.ds(v * 16, 16)], mask=m)
        plsc.store_scatter(cx1, [dst], x1_v[pl.ds(v * 16, 16)], mask=m)
        plsc.store_scatter(car, [dst], ar_v[pl.ds(v * 16, 16)], mask=m)
        sti[pl.ds(0, 16)] = basev + plsc.all_reduce_population_count(m)
        sti[pl.ds(16, 16)] = gi + sixteen
        return 0

    lax.fori_loop(0, _NVEC, comp_body, 0)
    nval = jnp.max(sti[pl.ds(0, 16)])
    nvec = (nval + 15) // 16

    def argmax_pass():
        stf[pl.ds(0, 16)] = jnp.full((16,), -2.0, jnp.float32)
        sti[pl.ds(32, 16)] = jnp.zeros((16,), jnp.int32)
        sti[pl.ds(16, 16)] = iota

        def abody(v, _):
            am = stf[pl.ds(0, 16)]
            ai = sti[pl.ds(32, 16)]
            iv = sti[pl.ds(16, 16)]
            sv = cs[pl.ds(v * 16, 16)]
            upd = sv > am
            stf[pl.ds(0, 16)] = jnp.where(upd, sv, am)
            sti[pl.ds(32, 16)] = jnp.where(upd, iv, ai)
            sti[pl.ds(16, 16)] = iv + sixteen
            return 0

        lax.fori_loop(0, nvec, abody, 0)
        am = stf[pl.ds(0, 16)]
        ai = sti[pl.ds(32, 16)]
        m = jnp.max(am)
        mf = jnp.full((16,), m, jnp.float32)
        ii = jnp.min(jnp.where(am >= mf, ai, big))
        stf[pl.ds(16, 16)] = mf
        sti[pl.ds(48, 16)] = jnp.full((16,), ii, jnp.int32)
        return m

    def cond(m):
        return m > 0.0

    def wbody(m):
        posv = sti[pl.ds(48, 16)]
        mv = stf[pl.ds(16, 16)]
        ky0 = plsc.load_gather(cy0, [posv])
        kx0 = plsc.load_gather(cx0, [posv])
        ky1 = plsc.load_gather(cy1, [posv])
        kx1 = plsc.load_gather(cx1, [posv])
        ka = plsc.load_gather(car, [posv])
        kidx = plsc.load_gather(ci, [posv])

        b5 = kidx * 5
        plsc.store_scatter(slab, [b5], ky0, mask=lane0)
        plsc.store_scatter(slab, [b5 + 1], kx0, mask=lane0)
        plsc.store_scatter(slab, [b5 + 2], ky1, mask=lane0)
        plsc.store_scatter(slab, [b5 + 3], kx1, mask=lane0)
        plsc.store_scatter(slab, [b5 + 4], mv, mask=lane0)
        plsc.store_scatter(cs, [posv], neg, mask=lane0)

        def sup(v, _):
            sl = pl.ds(v * 16, 16)
            ty = jnp.maximum(ky0, cy0[sl])
            tx = jnp.maximum(kx0, cx0[sl])
            by = jnp.minimum(ky1, cy1[sl])
            bx = jnp.minimum(kx1, cx1[sl])
            hh = jnp.maximum(by - ty, 0.0)
            ww = jnp.maximum(bx - tx, 0.0)
            inter = hh * ww
            denom = (ka + car[sl]) - inter + 1e-9
            kill = inter > _NMS_T * denom
            cs[sl] = jnp.where(kill, -1.0, cs[sl])
            return 0

        lax.fori_loop(0, nvec, sup, 0)
        return argmax_pass()

    m0 = argmax_pass()
    lax.while_loop(cond, wbody, m0)

    @pl.when(w < _NCLS)
    def _write():
        pltpu.sync_copy(slab, out_hbm.at[w])


def _sc_nms(probt, y0, x0, y1, x1, ar):
    mesh = plsc.VectorSubcoreMesh(core_axis_name="c", subcore_axis_name="s")
    f = pl.kernel(
        _sc_nms_body,
        mesh=mesh,
        compiler_params=pltpu.CompilerParams(needs_layout_passes=False),
        out_type=jax.ShapeDtypeStruct((_NCLS, _SLAB), jnp.float32),
        scratch_types=[
            pltpu.VMEM((_NPAD,), jnp.float32),
            pltpu.VMEM((_NPAD,), jnp.float32),
            pltpu.VMEM((_NPAD,), jnp.float32),
            pltpu.VMEM((_NPAD,), jnp.float32),
            pltpu.VMEM((_NPAD,), jnp.float32),
            pltpu.VMEM((_NPAD,), jnp.float32),
            pltpu.VMEM((_NPAD,), jnp.int32),
            pltpu.VMEM((_NPAD,), jnp.float32),
            pltpu.VMEM((_NPAD,), jnp.float32),
            pltpu.VMEM((_NPAD,), jnp.float32),
            pltpu.VMEM((_NPAD,), jnp.float32),
            pltpu.VMEM((_NPAD,), jnp.float32),
            pltpu.VMEM((_NPAD,), jnp.float32),
            pltpu.VMEM((_SLAB,), jnp.float32),
            pltpu.VMEM((32,), jnp.float32),
            pltpu.VMEM((64,), jnp.int32),
        ],
    )
    return f(probt, y0, x0, y1, x1, ar)


def kernel(roi_locs, roi_scores, rois):
    n = rois.shape[0]
    std = jnp.asarray([0.1, 0.1, 0.2, 0.2], jnp.float32)[None, :]
    loc = roi_locs[:, 4:8] * std
    h = rois[:, 2] - rois[:, 0]
    w = rois[:, 3] - rois[:, 1]
    cy = rois[:, 0] + 0.5 * h
    cx = rois[:, 1] + 0.5 * w
    ncy = loc[:, 0] * h + cy
    ncx = loc[:, 1] * w + cx
    nh = jnp.exp(loc[:, 2]) * h
    nw = jnp.exp(loc[:, 3]) * w
    y0 = jnp.clip(ncy - 0.5 * nh, 0.0, _IMG_H)
    x0 = jnp.clip(ncx - 0.5 * nw, 0.0, _IMG_W)
    y1 = jnp.clip(ncy + 0.5 * nh, 0.0, _IMG_H)
    x1 = jnp.clip(ncx + 0.5 * nw, 0.0, _IMG_W)
    area = jnp.clip(y1 - y0, 0.0) * jnp.clip(x1 - x0, 0.0)

    prob = jax.nn.softmax(roi_scores, axis=1)
    probt = jnp.full((_NW, _NPAD), -1.0, jnp.float32)
    probt = probt.at[:_NCLS, :n].set(prob.T[1:_NCLS + 1])

    pad0 = lambda v: jnp.pad(v, (0, _NPAD - n))
    out = _sc_nms(probt, pad0(y0), pad0(x0), pad0(y1), pad0(x1), pad0(area))
    return out[:, :n * 5].reshape(_NCLS, n, 5)


# speedup vs baseline: 201.4818x; 2.1795x over previous
import functools

import jax
import jax.numpy as jnp
from jax import lax
from jax.experimental import pallas as pl
from jax.experimental.pallas import tpu as pltpu
from jax.experimental.pallas import tpu_sc as plsc

_N = 5000
_NPAD = 5120
_NVEC = _NPAD // 16
_NCLS = 20
_NW = 32
_SLAB = 25600
_NMS_T = 0.3
_SCORE_T = 0.05
_NC = 2
_IMG_H, _IMG_W = 600.0, 800.0


def _sc_nms_body(probt_hbm, py0_h, px0_h, py1_h, px1_h, par_h, out_hbm,
                 prob_v, y0_v, x0_v, y1_v, x1_v, ar_v,
                 ci, cs, cy0, cx0, cy1, cx1, car, slab, stf, sti):
    w = lax.axis_index("s") * _NC + lax.axis_index("c")
    iota = lax.iota(jnp.int32, 16)
    lane0 = iota == 0
    zf = jnp.zeros((16,), jnp.float32)
    neg = jnp.full((16,), -1.0, jnp.float32)
    sixteen = jnp.full((16,), 16, jnp.int32)
    big = jnp.full((16,), 2 ** 30, jnp.int32)

    pltpu.sync_copy(probt_hbm.at[w], prob_v)
    pltpu.sync_copy(py0_h, y0_v)
    pltpu.sync_copy(px0_h, x0_v)
    pltpu.sync_copy(py1_h, y1_v)
    pltpu.sync_copy(px1_h, x1_v)
    pltpu.sync_copy(par_h, ar_v)

    def zero_body(v, _):
        slab[pl.ds(v * 16, 16)] = zf
        return 0

    lax.fori_loop(0, _SLAB // 16, zero_body, 0)

    def sent_body(v, _):
        cs[pl.ds(v * 16, 16)] = neg
        return 0

    lax.fori_loop(0, _NVEC, sent_body, 0)

    sti[pl.ds(0, 16)] = jnp.zeros((16,), jnp.int32)
    sti[pl.ds(16, 16)] = iota

    def comp_body(v, _):
        p = prob_v[pl.ds(v * 16, 16)]
        m = p > _SCORE_T
        pos = plsc.cumsum(m.astype(jnp.int32))
        basev = sti[pl.ds(0, 16)]
        dst = basev + pos - 1
        gi = sti[pl.ds(16, 16)]
        plsc.store_scatter(ci, [dst], gi, mask=m)
        plsc.store_scatter(cs, [dst], p, mask=m)
        plsc.store_scatter(cy0, [dst], y0_v[pl.ds(v * 16, 16)], mask=m)
        plsc.store_scatter(cx0, [dst], x0_v[pl.ds(v * 16, 16)], mask=m)
        plsc.store_scatter(cy1, [dst], y1_v[pl.ds(v * 16, 16)], mask=m)
        plsc.store_scatter(cx1, [dst], x1_v[pl.ds(v * 16, 16)], mask=m)
        plsc.store_scatter(car, [dst], ar_v[pl.ds(v * 16, 16)], mask=m)
        sti[pl.ds(0, 16)] = basev + plsc.all_reduce_population_count(m)
        sti[pl.ds(16, 16)] = gi + sixteen
        return 0

    lax.fori_loop(0, _NVEC, comp_body, 0)
    sti[pl.ds(64, 16)] = sti[pl.ds(0, 16)]

    ones = jnp.full((16,), 1, jnp.int32)
    zeros_i = jnp.zeros((16,), jnp.int32)

    def finish_argmax():
        am = stf[pl.ds(0, 16)]
        ai = sti[pl.ds(32, 16)]
        m = jnp.max(am)
        mf = jnp.full((16,), m, jnp.float32)
        ii = jnp.min(jnp.where(am >= mf, ai, big))
        stf[pl.ds(16, 16)] = mf
        sti[pl.ds(48, 16)] = jnp.full((16,), ii, jnp.int32)
        return m

    def argmax_pass(nvec):
        stf[pl.ds(0, 16)] = jnp.full((16,), -2.0, jnp.float32)
        sti[pl.ds(32, 16)] = zeros_i
        sti[pl.ds(16, 16)] = iota

        def abody(v, _):
            am = stf[pl.ds(0, 16)]
            ai = sti[pl.ds(32, 16)]
            iv = sti[pl.ds(16, 16)]
            sv = cs[pl.ds(v * 16, 16)]
            upd = sv > am
            stf[pl.ds(0, 16)] = jnp.where(upd, sv, am)
            sti[pl.ds(32, 16)] = jnp.where(upd, iv, ai)
            sti[pl.ds(16, 16)] = iv + sixteen
            return 0

        lax.fori_loop(0, nvec, abody, 0)
        return finish_argmax()

    def cond(carry):
        return carry[0] > 0.0

    def wbody(carry):
        _, it = carry
        posv = sti[pl.ds(48, 16)]
        mv = stf[pl.ds(16, 16)]
        ky0 = plsc.load_gather(cy0, [posv])
        kx0 = plsc.load_gather(cx0, [posv])
        ky1 = plsc.load_gather(cy1, [posv])
        kx1 = plsc.load_gather(cx1, [posv])
        ka = plsc.load_gather(car, [posv])
        kidx = plsc.load_gather(ci, [posv])

        b5 = kidx * 5
        plsc.store_scatter(slab, [b5], ky0, mask=lane0)
        plsc.store_scatter(slab, [b5 + 1], kx0, mask=lane0)
        plsc.store_scatter(slab, [b5 + 2], ky1, mask=lane0)
        plsc.store_scatter(slab, [b5 + 3], kx1, mask=lane0)
        plsc.store_scatter(slab, [b5 + 4], mv, mask=lane0)
        plsc.store_scatter(cs, [posv], neg, mask=lane0)

        @pl.when((it & 31) == 31)
        def _recompact():
            oldnv = (jnp.max(sti[pl.ds(64, 16)]) + 15) // 16
            sti[pl.ds(0, 16)] = zeros_i

            def rb(v, _):
                sl = pl.ds(v * 16, 16)
                sv = cs[sl]
                km = sv > 0.0
                pos = plsc.cumsum(jnp.where(km, ones, zeros_i))
                basev = sti[pl.ds(0, 16)]
                dst = basev + pos - 1
                plsc.store_scatter(ci, [dst], ci[sl], mask=km)
                plsc.store_scatter(cs, [dst], sv, mask=km)
                plsc.store_scatter(cy0, [dst], cy0[sl], mask=km)
                plsc.store_scatter(cx0, [dst], cx0[sl], mask=km)
                plsc.store_scatter(cy1, [dst], cy1[sl], mask=km)
                plsc.store_scatter(cx1, [dst], cx1[sl], mask=km)
                plsc.store_scatter(car, [dst], car[sl], mask=km)
                sti[pl.ds(0, 16)] = basev + \
                    plsc.all_reduce_population_count(km)
                return 0

            lax.fori_loop(0, oldnv, rb, 0)
            newv = jnp.max(sti[pl.ds(0, 16)])
            v0 = newv // 16
            rem = newv - v0 * 16
            bm = iota >= jnp.full((16,), rem, jnp.int32)
            bsl = pl.ds(v0 * 16, 16)
            cs[bsl] = jnp.where(bm, neg, cs[bsl])

            def fb(v, _):
                cs[pl.ds(v * 16, 16)] = neg
                return 0

            lax.fori_loop(v0 + 1, oldnv, fb, 0)
            sti[pl.ds(64, 16)] = jnp.full((16,), newv, jnp.int32)

        nvec = (jnp.max(sti[pl.ds(64, 16)]) + 15) // 16

        stf[pl.ds(0, 16)] = jnp.full((16,), -2.0, jnp.float32)
        sti[pl.ds(32, 16)] = zeros_i
        sti[pl.ds(16, 16)] = iota

        def sup(v, _):
            sl = pl.ds(v * 16, 16)
            sv = cs[sl]
            ty = jnp.maximum(ky0, cy0[sl])
            tx = jnp.maximum(kx0, cx0[sl])
            by = jnp.minimum(ky1, cy1[sl])
            bx = jnp.minimum(kx1, cx1[sl])
            hh = jnp.maximum(by - ty, 0.0)
            ww = jnp.maximum(bx - tx, 0.0)
            inter = hh * ww
            denom = (ka + car[sl]) - inter + 1e-9
            kill = inter > _NMS_T * denom
            nsv = jnp.where(kill, -1.0, sv)
            cs[sl] = nsv
            am = stf[pl.ds(0, 16)]
            ai = sti[pl.ds(32, 16)]
            iv = sti[pl.ds(16, 16)]
            upd = nsv > am
            stf[pl.ds(0, 16)] = jnp.where(upd, nsv, am)
            sti[pl.ds(32, 16)] = jnp.where(upd, iv, ai)
            sti[pl.ds(16, 16)] = iv + sixteen
            return 0

        lax.fori_loop(0, nvec, sup, 0)
        return finish_argmax(), it + 1

    m0 = argmax_pass((jnp.max(sti[pl.ds(64, 16)]) + 15) // 16)
    lax.while_loop(cond, wbody, (m0, jnp.int32(0)))

    @pl.when(w < _NCLS)
    def _write():
        pltpu.sync_copy(slab, out_hbm.at[w])


def _sc_nms(probt, y0, x0, y1, x1, ar):
    mesh = plsc.VectorSubcoreMesh(core_axis_name="c", subcore_axis_name="s")
    f = pl.kernel(
        _sc_nms_body,
        mesh=mesh,
        compiler_params=pltpu.CompilerParams(needs_layout_passes=False),
        out_type=jax.ShapeDtypeStruct((_NCLS, _SLAB), jnp.float32),
        scratch_types=[
            pltpu.VMEM((_NPAD,), jnp.float32),
            pltpu.VMEM((_NPAD,), jnp.float32),
            pltpu.VMEM((_NPAD,), jnp.float32),
            pltpu.VMEM((_NPAD,), jnp.float32),
            pltpu.VMEM((_NPAD,), jnp.float32),
            pltpu.VMEM((_NPAD,), jnp.float32),
            pltpu.VMEM((_NPAD,), jnp.int32),
            pltpu.VMEM((_NPAD,), jnp.float32),
            pltpu.VMEM((_NPAD,), jnp.float32),
            pltpu.VMEM((_NPAD,), jnp.float32),
            pltpu.VMEM((_NPAD,), jnp.float32),
            pltpu.VMEM((_NPAD,), jnp.float32),
            pltpu.VMEM((_NPAD,), jnp.float32),
            pltpu.VMEM((_SLAB,), jnp.float32),
            pltpu.VMEM((32,), jnp.float32),
            pltpu.VMEM((96,), jnp.int32),
        ],
    )
    return f(probt, y0, x0, y1, x1, ar)


def kernel(roi_locs, roi_scores, rois):
    n = rois.shape[0]
    std = jnp.asarray([0.1, 0.1, 0.2, 0.2], jnp.float32)[None, :]
    loc = roi_locs[:, 4:8] * std
    h = rois[:, 2] - rois[:, 0]
    w = rois[:, 3] - rois[:, 1]
    cy = rois[:, 0] + 0.5 * h
    cx = rois[:, 1] + 0.5 * w
    ncy = loc[:, 0] * h + cy
    ncx = loc[:, 1] * w + cx
    nh = jnp.exp(loc[:, 2]) * h
    nw = jnp.exp(loc[:, 3]) * w
    y0 = jnp.clip(ncy - 0.5 * nh, 0.0, _IMG_H)
    x0 = jnp.clip(ncx - 0.5 * nw, 0.0, _IMG_W)
    y1 = jnp.clip(ncy + 0.5 * nh, 0.0, _IMG_H)
    x1 = jnp.clip(ncx + 0.5 * nw, 0.0, _IMG_W)
    area = jnp.clip(y1 - y0, 0.0) * jnp.clip(x1 - x0, 0.0)

    prob = jax.nn.softmax(roi_scores, axis=1)
    probt = jnp.full((_NW, _NPAD), -1.0, jnp.float32)
    probt = probt.at[:_NCLS, :n].set(prob.T[1:_NCLS + 1])

    pad0 = lambda v: jnp.pad(v, (0, _NPAD - n))
    out = _sc_nms(probt, pad0(y0), pad0(x0), pad0(y1), pad0(x1), pad0(area))
    return out[:, :n * 5].reshape(_NCLS, n, 5)


# speedup vs baseline: 267.9674x; 1.3300x over previous
import functools

import jax
import jax.numpy as jnp
from jax import lax
from jax.experimental import pallas as pl
from jax.experimental.pallas import tpu as pltpu
from jax.experimental.pallas import tpu_sc as plsc

_N = 5000
_NPAD = 5120
_NVEC = _NPAD // 16
_NCLS = 20
_NW = 32
_SLAB = 25600
_NMS_T = 0.3
_SCORE_T = 0.05
_NC = 2
_IMG_H, _IMG_W = 600.0, 800.0


def _sc_nms_body(probt_hbm, py0_h, px0_h, py1_h, px1_h, par_h, out_hbm,
                 prob_v, y0_v, x0_v, y1_v, x1_v, ar_v,
                 ci, cs, cy0, cx0, cy1, cx1, car, slab, stf, sti):
    w = lax.axis_index("s") * _NC + lax.axis_index("c")
    iota = lax.iota(jnp.int32, 16)
    lane0 = iota == 0
    zf = jnp.zeros((16,), jnp.float32)
    neg = jnp.full((16,), -1.0, jnp.float32)
    sixteen = jnp.full((16,), 16, jnp.int32)
    big = jnp.full((16,), 2 ** 30, jnp.int32)

    pltpu.sync_copy(probt_hbm.at[w], prob_v)
    pltpu.sync_copy(py0_h, y0_v)
    pltpu.sync_copy(px0_h, x0_v)
    pltpu.sync_copy(py1_h, y1_v)
    pltpu.sync_copy(px1_h, x1_v)
    pltpu.sync_copy(par_h, ar_v)

    def zero_body(v, _):
        slab[pl.ds(v * 16, 16)] = zf
        return 0

    lax.fori_loop(0, _SLAB // 16, zero_body, 0)

    def sent_body(v, _):
        cs[pl.ds(v * 16, 16)] = neg
        return 0

    lax.fori_loop(0, _NVEC, sent_body, 0)

    ones = jnp.full((16,), 1, jnp.int32)
    zeros_i = jnp.zeros((16,), jnp.int32)

    def comp_body(v, acc):
        basev, gi = acc
        p = prob_v[pl.ds(v * 16, 16)]
        m = p > _SCORE_T
        pos = plsc.cumsum(jnp.where(m, ones, zeros_i))
        dst = basev + pos - 1
        plsc.store_scatter(ci, [dst], gi, mask=m)
        plsc.store_scatter(cs, [dst], p, mask=m)
        plsc.store_scatter(cy0, [dst], y0_v[pl.ds(v * 16, 16)], mask=m)
        plsc.store_scatter(cx0, [dst], x0_v[pl.ds(v * 16, 16)], mask=m)
        plsc.store_scatter(cy1, [dst], y1_v[pl.ds(v * 16, 16)], mask=m)
        plsc.store_scatter(cx1, [dst], x1_v[pl.ds(v * 16, 16)], mask=m)
        plsc.store_scatter(car, [dst], ar_v[pl.ds(v * 16, 16)], mask=m)
        return (basev + plsc.all_reduce_population_count(m), gi + sixteen)

    base0, _ = lax.fori_loop(0, _NVEC, comp_body, (zeros_i, iota))
    sti[pl.ds(64, 16)] = base0

    am0 = jnp.full((16,), -2.0, jnp.float32)

    def finish_argmax(am, ai):
        m = jnp.max(am)
        mf = jnp.full((16,), m, jnp.float32)
        ii = jnp.min(jnp.where(am >= mf, ai, big))
        stf[pl.ds(16, 16)] = mf
        sti[pl.ds(48, 16)] = jnp.full((16,), ii, jnp.int32)
        return m

    def argmax_pass(nvec):
        def abody(v, acc):
            am, ai, iv = acc
            sv = cs[pl.ds(v * 16, 16)]
            upd = sv > am
            return (jnp.where(upd, sv, am), jnp.where(upd, iv, ai),
                    iv + sixteen)

        am, ai, _ = lax.fori_loop(0, nvec, abody, (am0, zeros_i, iota))
        return finish_argmax(am, ai)

    def cond(carry):
        return carry[0] > 0.0

    def wbody(carry):
        _, it = carry
        posv = sti[pl.ds(48, 16)]
        mv = stf[pl.ds(16, 16)]
        ky0 = plsc.load_gather(cy0, [posv])
        kx0 = plsc.load_gather(cx0, [posv])
        ky1 = plsc.load_gather(cy1, [posv])
        kx1 = plsc.load_gather(cx1, [posv])
        ka = plsc.load_gather(car, [posv])
        kidx = plsc.load_gather(ci, [posv])

        b5 = kidx * 5
        plsc.store_scatter(slab, [b5], ky0, mask=lane0)
        plsc.store_scatter(slab, [b5 + 1], kx0, mask=lane0)
        plsc.store_scatter(slab, [b5 + 2], ky1, mask=lane0)
        plsc.store_scatter(slab, [b5 + 3], kx1, mask=lane0)
        plsc.store_scatter(slab, [b5 + 4], mv, mask=lane0)
        plsc.store_scatter(cs, [posv], neg, mask=lane0)

        @pl.when((it & 31) == 31)
        def _recompact():
            oldnv = (jnp.max(sti[pl.ds(64, 16)]) + 15) // 16

            def rb(v, basev):
                sl = pl.ds(v * 16, 16)
                sv = cs[sl]
                km = sv > 0.0
                pos = plsc.cumsum(jnp.where(km, ones, zeros_i))
                dst = basev + pos - 1
                plsc.store_scatter(ci, [dst], ci[sl], mask=km)
                plsc.store_scatter(cs, [dst], sv, mask=km)
                plsc.store_scatter(cy0, [dst], cy0[sl], mask=km)
                plsc.store_scatter(cx0, [dst], cx0[sl], mask=km)
                plsc.store_scatter(cy1, [dst], cy1[sl], mask=km)
                plsc.store_scatter(cx1, [dst], cx1[sl], mask=km)
                plsc.store_scatter(car, [dst], car[sl], mask=km)
                return basev + plsc.all_reduce_population_count(km)

            endbase = lax.fori_loop(0, oldnv, rb, zeros_i)
            newv = jnp.max(endbase)
            v0 = newv // 16
            rem = newv - v0 * 16
            bm = iota >= jnp.full((16,), rem, jnp.int32)
            bsl = pl.ds(v0 * 16, 16)
            cs[bsl] = jnp.where(bm, neg, cs[bsl])

            def fb(v, _):
                cs[pl.ds(v * 16, 16)] = neg
                return 0

            lax.fori_loop(v0 + 1, oldnv, fb, 0)
            sti[pl.ds(64, 16)] = jnp.full((16,), newv, jnp.int32)

        nvec = (jnp.max(sti[pl.ds(64, 16)]) + 15) // 16

        def sup(v, acc):
            am, ai, iv = acc
            sl = pl.ds(v * 16, 16)
            sv = cs[sl]
            ty = jnp.maximum(ky0, cy0[sl])
            tx = jnp.maximum(kx0, cx0[sl])
            by = jnp.minimum(ky1, cy1[sl])
            bx = jnp.minimum(kx1, cx1[sl])
            hh = jnp.maximum(by - ty, 0.0)
            ww = jnp.maximum(bx - tx, 0.0)
            inter = hh * ww
            denom = (ka + car[sl]) - inter + 1e-9
            kill = inter > _NMS_T * denom
            nsv = jnp.where(kill, -1.0, sv)
            cs[sl] = nsv
            upd = nsv > am
            return (jnp.where(upd, nsv, am), jnp.where(upd, iv, ai),
                    iv + sixteen)

        am, ai, _ = lax.fori_loop(0, nvec, sup, (am0, zeros_i, iota))
        return finish_argmax(am, ai), it + 1

    m0 = argmax_pass((jnp.max(sti[pl.ds(64, 16)]) + 15) // 16)
    lax.while_loop(cond, wbody, (m0, jnp.int32(0)))

    @pl.when(w < _NCLS)
    def _write():
        pltpu.sync_copy(slab, out_hbm.at[w])


def _sc_nms(probt, y0, x0, y1, x1, ar):
    mesh = plsc.VectorSubcoreMesh(core_axis_name="c", subcore_axis_name="s")
    f = pl.kernel(
        _sc_nms_body,
        mesh=mesh,
        compiler_params=pltpu.CompilerParams(needs_layout_passes=False),
        out_type=jax.ShapeDtypeStruct((_NCLS, _SLAB), jnp.float32),
        scratch_types=[
            pltpu.VMEM((_NPAD,), jnp.float32),
            pltpu.VMEM((_NPAD,), jnp.float32),
            pltpu.VMEM((_NPAD,), jnp.float32),
            pltpu.VMEM((_NPAD,), jnp.float32),
            pltpu.VMEM((_NPAD,), jnp.float32),
            pltpu.VMEM((_NPAD,), jnp.float32),
            pltpu.VMEM((_NPAD,), jnp.int32),
            pltpu.VMEM((_NPAD,), jnp.float32),
            pltpu.VMEM((_NPAD,), jnp.float32),
            pltpu.VMEM((_NPAD,), jnp.float32),
            pltpu.VMEM((_NPAD,), jnp.float32),
            pltpu.VMEM((_NPAD,), jnp.float32),
            pltpu.VMEM((_NPAD,), jnp.float32),
            pltpu.VMEM((_SLAB,), jnp.float32),
            pltpu.VMEM((32,), jnp.float32),
            pltpu.VMEM((96,), jnp.int32),
        ],
    )
    return f(probt, y0, x0, y1, x1, ar)


def kernel(roi_locs, roi_scores, rois):
    n = rois.shape[0]
    std = jnp.asarray([0.1, 0.1, 0.2, 0.2], jnp.float32)[None, :]
    loc = roi_locs[:, 4:8] * std
    h = rois[:, 2] - rois[:, 0]
    w = rois[:, 3] - rois[:, 1]
    cy = rois[:, 0] + 0.5 * h
    cx = rois[:, 1] + 0.5 * w
    ncy = loc[:, 0] * h + cy
    ncx = loc[:, 1] * w + cx
    nh = jnp.exp(loc[:, 2]) * h
    nw = jnp.exp(loc[:, 3]) * w
    y0 = jnp.clip(ncy - 0.5 * nh, 0.0, _IMG_H)
    x0 = jnp.clip(ncx - 0.5 * nw, 0.0, _IMG_W)
    y1 = jnp.clip(ncy + 0.5 * nh, 0.0, _IMG_H)
    x1 = jnp.clip(ncx + 0.5 * nw, 0.0, _IMG_W)
    area = jnp.clip(y1 - y0, 0.0) * jnp.clip(x1 - x0, 0.0)

    prob = jax.nn.softmax(roi_scores, axis=1)
    probt = jnp.full((_NW, _NPAD), -1.0, jnp.float32)
    probt = probt.at[:_NCLS, :n].set(prob.T[1:_NCLS + 1])

    pad0 = lambda v: jnp.pad(v, (0, _NPAD - n))
    out = _sc_nms(probt, pad0(y0), pad0(x0), pad0(y1), pad0(x1), pad0(area))
    return out[:, :n * 5].reshape(_NCLS, n, 5)


# speedup vs baseline: 487.8099x; 1.8204x over previous
import functools

import jax
import jax.numpy as jnp
from jax import lax
from jax.experimental import pallas as pl
from jax.experimental.pallas import tpu as pltpu
from jax.experimental.pallas import tpu_sc as plsc

_N = 5000
_NPAD = 5120
_NVEC = _NPAD // 16
_NCLS = 20
_NW = 32
_SLAB = 25600
_NMS_T = 0.3
_SCORE_T = 0.05
_NC = 2
_IMG_H, _IMG_W = 600.0, 800.0


def _sc_nms_body(probt_hbm, py0_h, px0_h, py1_h, px1_h, par_h, out_hbm,
                 prob_v, y0_v, x0_v, y1_v, x1_v, ar_v,
                 ci, cs, cy0, cx0, cy1, cx1, car, slab, stf, sti):
    w = lax.axis_index("s") * _NC + lax.axis_index("c")
    iota = lax.iota(jnp.int32, 16)
    lane0 = iota == 0
    zf = jnp.zeros((16,), jnp.float32)
    neg = jnp.full((16,), -1.0, jnp.float32)
    sixteen = jnp.full((16,), 16, jnp.int32)
    big = jnp.full((16,), 2 ** 30, jnp.int32)

    pltpu.sync_copy(probt_hbm.at[w], prob_v)
    pltpu.sync_copy(py0_h, y0_v)
    pltpu.sync_copy(px0_h, x0_v)
    pltpu.sync_copy(py1_h, y1_v)
    pltpu.sync_copy(px1_h, x1_v)
    pltpu.sync_copy(par_h, ar_v)

    def zero_body(v, _):
        slab[pl.ds(v * 16, 16)] = zf
        return 0

    lax.fori_loop(0, _SLAB // 16, zero_body, 0)

    def sent_body(v, _):
        cs[pl.ds(v * 16, 16)] = neg
        return 0

    lax.fori_loop(0, _NVEC, sent_body, 0)

    ones = jnp.full((16,), 1, jnp.int32)
    zeros_i = jnp.zeros((16,), jnp.int32)

    def comp_body(v, acc):
        basev, gi = acc
        p = prob_v[pl.ds(v * 16, 16)]
        m = p > _SCORE_T
        pos = plsc.cumsum(jnp.where(m, ones, zeros_i))
        dst = basev + pos - 1
        plsc.store_scatter(ci, [dst], gi, mask=m)
        plsc.store_scatter(cs, [dst], p, mask=m)
        plsc.store_scatter(cy0, [dst], y0_v[pl.ds(v * 16, 16)], mask=m)
        plsc.store_scatter(cx0, [dst], x0_v[pl.ds(v * 16, 16)], mask=m)
        plsc.store_scatter(cy1, [dst], y1_v[pl.ds(v * 16, 16)], mask=m)
        plsc.store_scatter(cx1, [dst], x1_v[pl.ds(v * 16, 16)], mask=m)
        plsc.store_scatter(car, [dst], ar_v[pl.ds(v * 16, 16)], mask=m)
        return (basev + plsc.all_reduce_population_count(m), gi + sixteen)

    base0, _ = lax.fori_loop(0, _NVEC, comp_body, (zeros_i, iota))
    sti[pl.ds(64, 16)] = base0

    am0 = jnp.full((16,), -2.0, jnp.float32)

    def finish_argmax(am, ai):
        m = jnp.max(am)
        mf = jnp.full((16,), m, jnp.float32)
        ii = jnp.min(jnp.where(am >= mf, ai, big))
        stf[pl.ds(16, 16)] = mf
        sti[pl.ds(48, 16)] = jnp.full((16,), ii, jnp.int32)
        return m

    def argmax_pass(nvec):
        def abody(v, acc):
            am, ai, iv = acc
            sv = cs[pl.ds(v * 16, 16)]
            upd = sv > am
            return (jnp.where(upd, sv, am), jnp.where(upd, iv, ai),
                    iv + sixteen)

        am, ai, _ = lax.fori_loop(0, nvec, abody, (am0, zeros_i, iota))
        return finish_argmax(am, ai)

    def cond(carry):
        return carry[0] > 0.0

    def wbody(carry):
        _, it = carry
        posv = sti[pl.ds(48, 16)]
        mv = stf[pl.ds(16, 16)]
        ky0 = plsc.load_gather(cy0, [posv])
        kx0 = plsc.load_gather(cx0, [posv])
        ky1 = plsc.load_gather(cy1, [posv])
        kx1 = plsc.load_gather(cx1, [posv])
        ka = plsc.load_gather(car, [posv])
        kidx = plsc.load_gather(ci, [posv])

        b5 = kidx * 5
        plsc.store_scatter(slab, [b5], ky0, mask=lane0)
        plsc.store_scatter(slab, [b5 + 1], kx0, mask=lane0)
        plsc.store_scatter(slab, [b5 + 2], ky1, mask=lane0)
        plsc.store_scatter(slab, [b5 + 3], kx1, mask=lane0)
        plsc.store_scatter(slab, [b5 + 4], mv, mask=lane0)
        plsc.store_scatter(cs, [posv], neg, mask=lane0)

        @pl.when((it & 31) == 31)
        def _recompact():
            oldnv = (jnp.max(sti[pl.ds(64, 16)]) + 15) // 16

            def rb(v, basev):
                sl = pl.ds(v * 16, 16)
                sv = cs[sl]
                km = sv > 0.0
                pos = plsc.cumsum(jnp.where(km, ones, zeros_i))
                dst = basev + pos - 1
                plsc.store_scatter(ci, [dst], ci[sl], mask=km)
                plsc.store_scatter(cs, [dst], sv, mask=km)
                plsc.store_scatter(cy0, [dst], cy0[sl], mask=km)
                plsc.store_scatter(cx0, [dst], cx0[sl], mask=km)
                plsc.store_scatter(cy1, [dst], cy1[sl], mask=km)
                plsc.store_scatter(cx1, [dst], cx1[sl], mask=km)
                plsc.store_scatter(car, [dst], car[sl], mask=km)
                return basev + plsc.all_reduce_population_count(km)

            endbase = lax.fori_loop(0, oldnv, rb, zeros_i)
            newv = jnp.max(endbase)
            v0 = newv // 16
            rem = newv - v0 * 16
            bm = iota >= jnp.full((16,), rem, jnp.int32)
            bsl = pl.ds(v0 * 16, 16)
            cs[bsl] = jnp.where(bm, neg, cs[bsl])

            def fb(v, _):
                cs[pl.ds(v * 16, 16)] = neg
                return 0

            lax.fori_loop(v0 + 1, oldnv, fb, 0)
            sti[pl.ds(64, 16)] = jnp.full((16,), newv, jnp.int32)

        nvec = (jnp.max(sti[pl.ds(64, 16)]) + 15) // 16

        @plsc.parallel_loop(0, nvec * 16, 16, unroll=4,
                            carry=(am0, zeros_i, iota))
        def sup(o, acc):
            am, ai, iv = acc
            sl = pl.ds(o, 16)
            sv = cs[sl]
            ty = jnp.maximum(ky0, cy0[sl])
            tx = jnp.maximum(kx0, cx0[sl])
            by = jnp.minimum(ky1, cy1[sl])
            bx = jnp.minimum(kx1, cx1[sl])
            hh = jnp.maximum(by - ty, 0.0)
            ww = jnp.maximum(bx - tx, 0.0)
            inter = hh * ww
            denom = (ka + car[sl]) - inter + 1e-9
            kill = inter > _NMS_T * denom
            nsv = jnp.where(kill, -1.0, sv)
            cs[sl] = nsv
            upd = nsv > am
            return (jnp.where(upd, nsv, am), jnp.where(upd, iv, ai),
                    iv + sixteen)

        am, ai, _ = sup
        return finish_argmax(am, ai), it + 1

    m0 = argmax_pass((jnp.max(sti[pl.ds(64, 16)]) + 15) // 16)
    lax.while_loop(cond, wbody, (m0, jnp.int32(0)))

    @pl.when(w < _NCLS)
    def _write():
        pltpu.sync_copy(slab, out_hbm.at[w])


def _sc_nms(probt, y0, x0, y1, x1, ar):
    mesh = plsc.VectorSubcoreMesh(core_axis_name="c", subcore_axis_name="s")
    f = pl.kernel(
        _sc_nms_body,
        mesh=mesh,
        compiler_params=pltpu.CompilerParams(needs_layout_passes=False),
        out_type=jax.ShapeDtypeStruct((_NCLS, _SLAB), jnp.float32),
        scratch_types=[
            pltpu.VMEM((_NPAD,), jnp.float32),
            pltpu.VMEM((_NPAD,), jnp.float32),
            pltpu.VMEM((_NPAD,), jnp.float32),
            pltpu.VMEM((_NPAD,), jnp.float32),
            pltpu.VMEM((_NPAD,), jnp.float32),
            pltpu.VMEM((_NPAD,), jnp.float32),
            pltpu.VMEM((_NPAD,), jnp.int32),
            pltpu.VMEM((_NPAD,), jnp.float32),
            pltpu.VMEM((_NPAD,), jnp.float32),
            pltpu.VMEM((_NPAD,), jnp.float32),
            pltpu.VMEM((_NPAD,), jnp.float32),
            pltpu.VMEM((_NPAD,), jnp.float32),
            pltpu.VMEM((_NPAD,), jnp.float32),
            pltpu.VMEM((_SLAB,), jnp.float32),
            pltpu.VMEM((32,), jnp.float32),
            pltpu.VMEM((96,), jnp.int32),
        ],
    )
    return f(probt, y0, x0, y1, x1, ar)


def kernel(roi_locs, roi_scores, rois):
    n = rois.shape[0]
    std = jnp.asarray([0.1, 0.1, 0.2, 0.2], jnp.float32)[None, :]
    loc = roi_locs[:, 4:8] * std
    h = rois[:, 2] - rois[:, 0]
    w = rois[:, 3] - rois[:, 1]
    cy = rois[:, 0] + 0.5 * h
    cx = rois[:, 1] + 0.5 * w
    ncy = loc[:, 0] * h + cy
    ncx = loc[:, 1] * w + cx
    nh = jnp.exp(loc[:, 2]) * h
    nw = jnp.exp(loc[:, 3]) * w
    y0 = jnp.clip(ncy - 0.5 * nh, 0.0, _IMG_H)
    x0 = jnp.clip(ncx - 0.5 * nw, 0.0, _IMG_W)
    y1 = jnp.clip(ncy + 0.5 * nh, 0.0, _IMG_H)
    x1 = jnp.clip(ncx + 0.5 * nw, 0.0, _IMG_W)
    area = jnp.clip(y1 - y0, 0.0) * jnp.clip(x1 - x0, 0.0)

    prob = jax.nn.softmax(roi_scores, axis=1)
    probt = jnp.full((_NW, _NPAD), -1.0, jnp.float32)
    probt = probt.at[:_NCLS, :n].set(prob.T[1:_NCLS + 1])

    pad0 = lambda v: jnp.pad(v, (0, _NPAD - n))
    out = _sc_nms(probt, pad0(y0), pad0(x0), pad0(y1), pad0(x1), pad0(area))
    return out[:, :n * 5].reshape(_NCLS, n, 5)


# speedup vs baseline: 490.0196x; 1.0045x over previous
import functools

import jax
import jax.numpy as jnp
from jax import lax
from jax.experimental import pallas as pl
from jax.experimental.pallas import tpu as pltpu
from jax.experimental.pallas import tpu_sc as plsc

_N = 5000
_NPAD = 5120
_NVEC = _NPAD // 16
_NCLS = 20
_NW = 32
_SLAB = 25600
_NMS_T = 0.3
_SCORE_T = 0.05
_NC = 2
_IMG_H, _IMG_W = 600.0, 800.0


def _sc_nms_body(probt_hbm, py0_h, px0_h, py1_h, px1_h, par_h, out_hbm,
                 prob_v, y0_v, x0_v, y1_v, x1_v, ar_v,
                 ci, cs, cy0, cx0, cy1, cx1, car, slab, stf, sti):
    w = lax.axis_index("s") * _NC + lax.axis_index("c")
    iota = lax.iota(jnp.int32, 16)
    lane0 = iota == 0
    zf = jnp.zeros((16,), jnp.float32)
    neg = jnp.full((16,), -1.0, jnp.float32)
    sixteen = jnp.full((16,), 16, jnp.int32)
    big = jnp.full((16,), 2 ** 30, jnp.int32)

    pltpu.sync_copy(probt_hbm.at[w], prob_v)
    pltpu.sync_copy(py0_h, y0_v)
    pltpu.sync_copy(px0_h, x0_v)
    pltpu.sync_copy(py1_h, y1_v)
    pltpu.sync_copy(px1_h, x1_v)
    pltpu.sync_copy(par_h, ar_v)

    @plsc.parallel_loop(0, _SLAB, 16, unroll=8)
    def _zero_body(o):
        slab[pl.ds(o, 16)] = zf

    @plsc.parallel_loop(0, _NPAD, 16, unroll=8)
    def _sent_body(o):
        cs[pl.ds(o, 16)] = neg

    ones = jnp.full((16,), 1, jnp.int32)
    zeros_i = jnp.zeros((16,), jnp.int32)

    @plsc.parallel_loop(0, _NPAD, 16, unroll=4, carry=(zeros_i, iota))
    def comp_body(o, acc):
        basev, gi = acc
        p = prob_v[pl.ds(o, 16)]
        m = p > _SCORE_T
        pos = plsc.cumsum(jnp.where(m, ones, zeros_i))
        dst = basev + pos - 1
        plsc.store_scatter(ci, [dst], gi, mask=m)
        plsc.store_scatter(cs, [dst], p, mask=m)
        plsc.store_scatter(cy0, [dst], y0_v[pl.ds(o, 16)], mask=m)
        plsc.store_scatter(cx0, [dst], x0_v[pl.ds(o, 16)], mask=m)
        plsc.store_scatter(cy1, [dst], y1_v[pl.ds(o, 16)], mask=m)
        plsc.store_scatter(cx1, [dst], x1_v[pl.ds(o, 16)], mask=m)
        plsc.store_scatter(car, [dst], ar_v[pl.ds(o, 16)], mask=m)
        return (basev + plsc.all_reduce_population_count(m), gi + sixteen)

    base0, _ = comp_body
    sti[pl.ds(64, 16)] = base0

    am0 = jnp.full((16,), -2.0, jnp.float32)

    def finish_argmax(am, ai):
        m = jnp.max(am)
        mf = jnp.full((16,), m, jnp.float32)
        ii = jnp.min(jnp.where(am >= mf, ai, big))
        stf[pl.ds(16, 16)] = mf
        sti[pl.ds(48, 16)] = jnp.full((16,), ii, jnp.int32)
        return m

    def argmax_pass(nvec):
        @plsc.parallel_loop(0, nvec * 16, 16, unroll=4,
                            carry=(am0, zeros_i, iota))
        def abody(o, acc):
            am, ai, iv = acc
            sv = cs[pl.ds(o, 16)]
            upd = sv > am
            return (jnp.where(upd, sv, am), jnp.where(upd, iv, ai),
                    iv + sixteen)

        am, ai, _ = abody
        return finish_argmax(am, ai)

    def cond(carry):
        return carry[0] > 0.0

    def wbody(carry):
        _, it = carry
        posv = sti[pl.ds(48, 16)]
        mv = stf[pl.ds(16, 16)]
        ky0 = plsc.load_gather(cy0, [posv])
        kx0 = plsc.load_gather(cx0, [posv])
        ky1 = plsc.load_gather(cy1, [posv])
        kx1 = plsc.load_gather(cx1, [posv])
        ka = plsc.load_gather(car, [posv])
        kidx = plsc.load_gather(ci, [posv])

        b5 = kidx * 5
        plsc.store_scatter(slab, [b5], ky0, mask=lane0)
        plsc.store_scatter(slab, [b5 + 1], kx0, mask=lane0)
        plsc.store_scatter(slab, [b5 + 2], ky1, mask=lane0)
        plsc.store_scatter(slab, [b5 + 3], kx1, mask=lane0)
        plsc.store_scatter(slab, [b5 + 4], mv, mask=lane0)
        plsc.store_scatter(cs, [posv], neg, mask=lane0)

        @pl.when((it & 31) == 31)
        def _recompact():
            oldnv = (jnp.max(sti[pl.ds(64, 16)]) + 15) // 16

            def rb(v, basev):
                sl = pl.ds(v * 16, 16)
                sv = cs[sl]
                km = sv > 0.0
                pos = plsc.cumsum(jnp.where(km, ones, zeros_i))
                dst = basev + pos - 1
                plsc.store_scatter(ci, [dst], ci[sl], mask=km)
                plsc.store_scatter(cs, [dst], sv, mask=km)
                plsc.store_scatter(cy0, [dst], cy0[sl], mask=km)
                plsc.store_scatter(cx0, [dst], cx0[sl], mask=km)
                plsc.store_scatter(cy1, [dst], cy1[sl], mask=km)
                plsc.store_scatter(cx1, [dst], cx1[sl], mask=km)
                plsc.store_scatter(car, [dst], car[sl], mask=km)
                return basev + plsc.all_reduce_population_count(km)

            endbase = lax.fori_loop(0, oldnv, rb, zeros_i)
            newv = jnp.max(endbase)
            v0 = newv // 16
            rem = newv - v0 * 16
            bm = iota >= jnp.full((16,), rem, jnp.int32)
            bsl = pl.ds(v0 * 16, 16)
            cs[bsl] = jnp.where(bm, neg, cs[bsl])

            def fb(v, _):
                cs[pl.ds(v * 16, 16)] = neg
                return 0

            lax.fori_loop(v0 + 1, oldnv, fb, 0)
            sti[pl.ds(64, 16)] = jnp.full((16,), newv, jnp.int32)

        nvec = (jnp.max(sti[pl.ds(64, 16)]) + 15) // 16

        @plsc.parallel_loop(0, nvec * 16, 16, unroll=8,
                            carry=(am0, zeros_i, iota))
        def sup(o, acc):
            am, ai, iv = acc
            sl = pl.ds(o, 16)
            sv = cs[sl]
            ty = jnp.maximum(ky0, cy0[sl])
            tx = jnp.maximum(kx0, cx0[sl])
            by = jnp.minimum(ky1, cy1[sl])
            bx = jnp.minimum(kx1, cx1[sl])
            hh = jnp.maximum(by - ty, 0.0)
            ww = jnp.maximum(bx - tx, 0.0)
            inter = hh * ww
            denom = (ka + car[sl]) - inter + 1e-9
            kill = inter > _NMS_T * denom
            nsv = jnp.where(kill, -1.0, sv)
            cs[sl] = nsv
            upd = nsv > am
            return (jnp.where(upd, nsv, am), jnp.where(upd, iv, ai),
                    iv + sixteen)

        am, ai, _ = sup
        return finish_argmax(am, ai), it + 1

    m0 = argmax_pass((jnp.max(sti[pl.ds(64, 16)]) + 15) // 16)
    lax.while_loop(cond, wbody, (m0, jnp.int32(0)))

    @pl.when(w < _NCLS)
    def _write():
        pltpu.sync_copy(slab, out_hbm.at[w])


def _sc_nms(probt, y0, x0, y1, x1, ar):
    mesh = plsc.VectorSubcoreMesh(core_axis_name="c", subcore_axis_name="s")
    f = pl.kernel(
        _sc_nms_body,
        mesh=mesh,
        compiler_params=pltpu.CompilerParams(needs_layout_passes=False),
        out_type=jax.ShapeDtypeStruct((_NCLS, _SLAB), jnp.float32),
        scratch_types=[
            pltpu.VMEM((_NPAD,), jnp.float32),
            pltpu.VMEM((_NPAD,), jnp.float32),
            pltpu.VMEM((_NPAD,), jnp.float32),
            pltpu.VMEM((_NPAD,), jnp.float32),
            pltpu.VMEM((_NPAD,), jnp.float32),
            pltpu.VMEM((_NPAD,), jnp.float32),
            pltpu.VMEM((_NPAD,), jnp.int32),
            pltpu.VMEM((_NPAD,), jnp.float32),
            pltpu.VMEM((_NPAD,), jnp.float32),
            pltpu.VMEM((_NPAD,), jnp.float32),
            pltpu.VMEM((_NPAD,), jnp.float32),
            pltpu.VMEM((_NPAD,), jnp.float32),
            pltpu.VMEM((_NPAD,), jnp.float32),
            pltpu.VMEM((_SLAB,), jnp.float32),
            pltpu.VMEM((32,), jnp.float32),
            pltpu.VMEM((96,), jnp.int32),
        ],
    )
    return f(probt, y0, x0, y1, x1, ar)


def kernel(roi_locs, roi_scores, rois):
    n = rois.shape[0]
    std = jnp.asarray([0.1, 0.1, 0.2, 0.2], jnp.float32)[None, :]
    loc = roi_locs[:, 4:8] * std
    h = rois[:, 2] - rois[:, 0]
    w = rois[:, 3] - rois[:, 1]
    cy = rois[:, 0] + 0.5 * h
    cx = rois[:, 1] + 0.5 * w
    ncy = loc[:, 0] * h + cy
    ncx = loc[:, 1] * w + cx
    nh = jnp.exp(loc[:, 2]) * h
    nw = jnp.exp(loc[:, 3]) * w
    y0 = jnp.clip(ncy - 0.5 * nh, 0.0, _IMG_H)
    x0 = jnp.clip(ncx - 0.5 * nw, 0.0, _IMG_W)
    y1 = jnp.clip(ncy + 0.5 * nh, 0.0, _IMG_H)
    x1 = jnp.clip(ncx + 0.5 * nw, 0.0, _IMG_W)
    area = jnp.clip(y1 - y0, 0.0) * jnp.clip(x1 - x0, 0.0)

    prob = jax.nn.softmax(roi_scores, axis=1)
    probt = jnp.full((_NW, _NPAD), -1.0, jnp.float32)
    probt = probt.at[:_NCLS, :n].set(prob.T[1:_NCLS + 1])

    pad0 = lambda v: jnp.pad(v, (0, _NPAD - n))
    out = _sc_nms(probt, pad0(y0), pad0(x0), pad0(y1), pad0(x1), pad0(area))
    return out[:, :n * 5].reshape(_NCLS, n, 5)
